# Initial kernel scaffold; baseline (speedup 1.0000x reference)
import jax
import jax.numpy as jnp
from jax.experimental import pallas as pl


def kernel(x, edge_index, edge_attr, batch, opt_edge, We, be, Weo, beo, Wl1, bl1, Wn1, bn1, Wl2, bl2, Wn2, bn2, Wl3, bl3, Wn3, bn3, Wf1, bf1, Wf2, bf2):
    raise NotImplementedError("write your pallas kernel here")


# speedup vs baseline: 4.3066x; 4.3066x over previous
import functools

import jax
import jax.numpy as jnp
from jax import lax
from jax.experimental import pallas as pl
from jax.experimental.pallas import tpu as pltpu
from jax.experimental.pallas import tpu_sc as plsc

N, E, G, H = 10000, 320000, 64, 128
CH = 128
NCHUNK = E // CH
ROWC = 80
NROWC = N // ROWC
NTPC = 16
NW = 32
f32 = jnp.float32
i32 = jnp.int32


def _mesh():
    return plsc.VectorSubcoreMesh(core_axis_name="c", subcore_axis_name="s")


def _k1_body(reci, recf, xf, coef, s_out, recbuf, fbuf, cbuf, xsrc, mbuf, aggr, sem):
    cid = lax.axis_index("c")
    sid = lax.axis_index("s")

    @pl.when(cid == 0)
    def _core0():
        pltpu.sync_copy(coef, cbuf)
        p1v = cbuf[0, :]
        q1v = cbuf[1, :]
        r1v = cbuf[2, :]

        def initb(i, carry):
            c = i * NTPC + sid

            @pl.when(c < NROWC)
            def _():
                pltpu.sync_copy(xf.at[pl.ds(c * ROWC, ROWC)],
                                mbuf.at[pl.ds(0, ROWC)])
                pltpu.sync_copy(mbuf.at[pl.ds(0, ROWC)],
                                aggr.at[pl.ds(c * ROWC, ROWC)])
            return carry

        lax.fori_loop(0, (NROWC + NTPC - 1) // NTPC, initb, 0)

        plsc.subcore_barrier()

        def chunk(i, carry):
            c = i * NTPC + sid

            @pl.when(c < NCHUNK)
            def _():
                base = c * CH
                pltpu.sync_copy(reci.at[:, pl.ds(base, CH)], recbuf)
                pltpu.sync_copy(recf.at[:, pl.ds(base, CH)], fbuf)
                pltpu.async_copy(xf.at[recbuf.at[0]], xsrc, sem).wait()
                for g in range(8):
                    sl = pl.ds(g * 16, 16)
                    av = fbuf[0, sl]
                    ov = fbuf[1, sl]
                    mbuf[sl] = jnp.maximum(
                        xsrc[sl] + av * p1v + ov * q1v + r1v, 0.0)
                pltpu.sync_copy(mbuf, aggr.at[recbuf.at[1]], add=True)
            return carry

        lax.fori_loop(0, (NCHUNK + NTPC - 1) // NTPC, chunk, 0)
        plsc.subcore_barrier()

        def wb(i, carry):
            c = i * NTPC + sid

            @pl.when(c < NROWC)
            def _():
                pltpu.sync_copy(aggr.at[pl.ds(c * ROWC, ROWC)],
                                mbuf.at[pl.ds(0, ROWC)])
                pltpu.sync_copy(mbuf.at[pl.ds(0, ROWC)],
                                s_out.at[pl.ds(c * ROWC, ROWC)])
            return carry

        lax.fori_loop(0, (NROWC + NTPC - 1) // NTPC, wb, 0)


def _k1(reci, recf, xf, coef):
    return pl.kernel(
        _k1_body,
        out_type=jax.ShapeDtypeStruct((N,), f32),
        mesh=_mesh(),
        scratch_types=[
            pltpu.VMEM((2, CH), i32),
            pltpu.VMEM((2, CH), f32),
            pltpu.VMEM((3, 16), f32),
            pltpu.VMEM((CH,), f32),
            pltpu.VMEM((CH,), f32),
            pltpu.VMEM_SHARED((N,), f32),
            pltpu.SemaphoreType.DMA,
        ],
    )(reci, recf, xf, coef)


def _k2_body(reci, recf, s_hbm, coef, p2, recbuf, fbuf, cbuf, ssrc, mbuf, aggr, sem):
    cid = lax.axis_index("c")
    sid = lax.axis_index("s")
    wid = cid * NTPC + sid

    pltpu.sync_copy(coef, cbuf)
    w1s = [cbuf[0, pl.ds(g * 16, 16)] for g in range(8)]
    u2s = [cbuf[1, pl.ds(g * 16, 16)] for g in range(8)]
    v2s = [cbuf[2, pl.ds(g * 16, 16)] for g in range(8)]
    ccs = [cbuf[3, pl.ds(g * 16, 16)] for g in range(8)]
    bns = [cbuf[4, pl.ds(g * 16, 16)] for g in range(8)]

    zv = jnp.zeros((16,), f32)

    def zb(i, carry):
        for g in range(8):
            mbuf[i, pl.ds(g * 16, 16)] = zv
        return carry

    lax.fori_loop(0, CH, zb, 0)

    def zc(i, carry):
        c = i * NTPC + sid

        @pl.when(c < NROWC)
        def _():
            pltpu.sync_copy(mbuf.at[pl.ds(0, ROWC)],
                            aggr.at[pl.ds(c * ROWC, ROWC)])
        return carry

    lax.fori_loop(0, (NROWC + NTPC - 1) // NTPC, zc, 0)

    plsc.subcore_barrier()

    def chunk(i, carry):
        c = i * NW + wid

        @pl.when(c < NCHUNK)
        def _():
            base = c * CH
            pltpu.sync_copy(reci.at[:, pl.ds(base, CH)], recbuf)
            pltpu.sync_copy(recf.at[:, pl.ds(base, CH)], fbuf)
            pltpu.async_copy(s_hbm.at[recbuf.at[0]], ssrc, sem).wait()

            def edgegrp(q, carry2):
                qb = q * 16
                sv = ssrc[pl.ds(qb, 16)]
                av = fbuf[0, pl.ds(qb, 16)]
                ov = fbuf[1, pl.ds(qb, 16)]
                for e in range(16):
                    se = sv[e]
                    ae = av[e]
                    oe = ov[e]
                    for g in range(8):
                        mbuf[qb + e, pl.ds(g * 16, 16)] = jnp.maximum(
                            se * w1s[g] + ae * u2s[g] + oe * v2s[g] + ccs[g],
                            0.0)
                return carry2

            lax.fori_loop(0, CH // 16, edgegrp, 0)
            pltpu.sync_copy(mbuf, aggr.at[recbuf.at[1]], add=True)
        return carry

    lax.fori_loop(0, (NCHUNK + NW - 1) // NW, chunk, 0)
    plsc.subcore_barrier()

    def _wb_rows(base, nrows, mslice):
        pltpu.sync_copy(aggr.at[pl.ds(base, nrows)], mslice)

        @pl.when(cid == 0)
        def _():
            pltpu.sync_copy(s_hbm.at[pl.ds(base, nrows)],
                            ssrc.at[pl.ds(0, nrows)])

            def rowgrp(q, carry):
                qb = q * 16
                sv = ssrc[pl.ds(qb, 16)]
                for e in range(16):
                    se = sv[e]
                    for g in range(8):
                        sl = pl.ds(g * 16, 16)
                        mbuf[qb + e, sl] = mbuf[qb + e, sl] + se * w1s[g] \
                            + bns[g]
                return carry

            lax.fori_loop(0, nrows // 16, rowgrp, 0)
        pltpu.sync_copy(mslice, p2.at[pl.ds(cid * N + base, nrows)])

    def wb(i, carry):
        c = i * NTPC + sid

        @pl.when(c < NROWC)
        def _():
            _wb_rows(c * ROWC, ROWC, mbuf.at[pl.ds(0, ROWC)])
        return carry

    lax.fori_loop(0, (NROWC + NTPC - 1) // NTPC, wb, 0)


def _k2(reci, recf, s, coef):
    return pl.kernel(
        _k2_body,
        out_type=jax.ShapeDtypeStruct((2 * N, H), f32),
        mesh=_mesh(),
        scratch_types=[
            pltpu.VMEM((2, CH), i32),
            pltpu.VMEM((2, CH), f32),
            pltpu.VMEM((5, H), f32),
            pltpu.VMEM((CH,), f32),
            pltpu.VMEM((CH, H), f32),
            pltpu.VMEM_SHARED((N, H), f32),
            pltpu.SemaphoreType.DMA,
        ],
    )(reci, recf, s, coef)


def _k4_body(reci, recf, h2_hbm, coef, p3, recbuf, fbuf, cbuf, hrows, h2b, aggr, sem):
    cid = lax.axis_index("c")
    sid = lax.axis_index("s")
    wid = cid * NTPC + sid

    pltpu.sync_copy(coef, cbuf)
    u3s = [cbuf[0, pl.ds(g * 16, 16)] for g in range(8)]
    v3s = [cbuf[1, pl.ds(g * 16, 16)] for g in range(8)]
    c3s = [cbuf[2, pl.ds(g * 16, 16)] for g in range(8)]

    zv = jnp.zeros((16,), f32)

    def zb(i, carry):
        for g in range(8):
            hrows[i, pl.ds(g * 16, 16)] = zv
        return carry

    lax.fori_loop(0, CH, zb, 0)

    def zc(i, carry):
        c = i * NTPC + sid

        @pl.when(c < NROWC)
        def _():
            pltpu.sync_copy(hrows.at[pl.ds(0, ROWC)],
                            aggr.at[pl.ds(c * ROWC, ROWC)])
        return carry

    lax.fori_loop(0, (NROWC + NTPC - 1) // NTPC, zc, 0)

    plsc.subcore_barrier()

    def chunk(i, carry):
        c = i * NW + wid

        @pl.when(c < NCHUNK)
        def _():
            base = c * CH
            pltpu.sync_copy(reci.at[:, pl.ds(base, CH)], recbuf)
            pltpu.sync_copy(recf.at[:, pl.ds(base, CH)], fbuf)
            pltpu.async_copy(h2_hbm.at[recbuf.at[0]], hrows, sem).wait()

            def edgegrp(q, carry2):
                qb = q * 16
                av = fbuf[0, pl.ds(qb, 16)]
                ov = fbuf[1, pl.ds(qb, 16)]
                for e in range(16):
                    ae = av[e]
                    oe = ov[e]
                    for g in range(8):
                        sl = pl.ds(g * 16, 16)
                        hrows[qb + e, sl] = jnp.maximum(
                            hrows[qb + e, sl] + ae * u3s[g] + oe * v3s[g]
                            + c3s[g], 0.0)
                return carry2

            lax.fori_loop(0, CH // 16, edgegrp, 0)
            pltpu.sync_copy(hrows, aggr.at[recbuf.at[1]], add=True)
        return carry

    lax.fori_loop(0, (NCHUNK + NW - 1) // NW, chunk, 0)
    plsc.subcore_barrier()

    def _wb_rows(base, nrows, hslice, h2slice):
        pltpu.sync_copy(aggr.at[pl.ds(base, nrows)], hslice)

        @pl.when(cid == 0)
        def _():
            pltpu.sync_copy(h2_hbm.at[pl.ds(base, nrows)], h2slice)

            def row(r, carry):
                for g in range(8):
                    sl = pl.ds(g * 16, 16)
                    hrows[r, sl] = hrows[r, sl] + h2b[r, sl]
                return carry

            lax.fori_loop(0, nrows, row, 0)
        pltpu.sync_copy(hslice, p3.at[pl.ds(cid * N + base, nrows)])

    def wb(i, carry):
        c = i * NTPC + sid

        @pl.when(c < NROWC)
        def _():
            _wb_rows(c * ROWC, ROWC, hrows.at[pl.ds(0, ROWC)],
                     h2b.at[pl.ds(0, ROWC)])
        return carry

    lax.fori_loop(0, (NROWC + NTPC - 1) // NTPC, wb, 0)


def _k4(reci, recf, h2, coef):
    return pl.kernel(
        _k4_body,
        out_type=jax.ShapeDtypeStruct((2 * N, H), f32),
        mesh=_mesh(),
        scratch_types=[
            pltpu.VMEM((2, CH), i32),
            pltpu.VMEM((2, CH), f32),
            pltpu.VMEM((3, H), f32),
            pltpu.VMEM((CH, H), f32),
            pltpu.VMEM((CH, H), f32),
            pltpu.VMEM_SHARED((N, H), f32),
            pltpu.SemaphoreType.DMA,
        ],
    )(reci, recf, h2, coef)


BLK = 1000


def _mm_body(pa, pb, w, b, o):
    acc = pa[...] + pb[...]
    o[...] = jnp.dot(acc, w[...], preferred_element_type=f32) + b[...]


def _mm(p, w, b):
    return pl.pallas_call(
        _mm_body,
        grid=(N // BLK,),
        in_specs=[
            pl.BlockSpec((BLK, H), lambda i: (i, 0)),
            pl.BlockSpec((BLK, H), lambda i: (i + N // BLK, 0)),
            pl.BlockSpec((H, H), lambda i: (0, 0)),
            pl.BlockSpec((1, H), lambda i: (0, 0)),
        ],
        out_specs=pl.BlockSpec((BLK, H), lambda i: (i, 0)),
        out_shape=jax.ShapeDtypeStruct((N, H), f32),
    )(p, p, w, b)


def _k5_body(pa, pb, w3, b3, bt, wf1, bf1, wf2, bf2, out, pooled):
    i = pl.program_id(0)

    @pl.when(i == 0)
    def _():
        pooled[...] = jnp.full((G, H), -jnp.inf, f32)

    h3 = jnp.dot(pa[...] + pb[...], w3[...],
                 preferred_element_type=f32) + b3[...]
    b = bt[0, 0, :]
    bmin = jnp.min(b)
    bmax = jnp.max(b)
    bc = b[:, None]
    for g in range(G):
        @pl.when((bmin <= g) & (g <= bmax))
        def _(g=g):
            cand = jnp.where(bc == g, h3, -jnp.inf)
            m = jnp.max(cand, axis=0, keepdims=True)
            pooled[pl.ds(g, 1), :] = jnp.maximum(pooled[pl.ds(g, 1), :], m)

    @pl.when(i == pl.num_programs(0) - 1)
    def _():
        p = pooled[...]
        e1 = jnp.maximum(
            jnp.dot(p, wf1[...], preferred_element_type=f32) + bf1[...], 0.0)
        out[...] = jnp.dot(e1, wf2[...], preferred_element_type=f32) + bf2[...]


def _k5(p, w3, b3, bt, wf1, bf1, wf2, bf2):
    nb = N // BLK
    return pl.pallas_call(
        _k5_body,
        grid=(nb,),
        in_specs=[
            pl.BlockSpec((BLK, H), lambda i: (i, 0)),
            pl.BlockSpec((BLK, H), lambda i: (i + nb, 0)),
            pl.BlockSpec((H, H), lambda i: (0, 0)),
            pl.BlockSpec((1, H), lambda i: (0, 0)),
            pl.BlockSpec((1, 1, BLK), lambda i: (i, 0, 0)),
            pl.BlockSpec((H, H), lambda i: (0, 0)),
            pl.BlockSpec((1, H), lambda i: (0, 0)),
            pl.BlockSpec((H, 1), lambda i: (0, 0)),
            pl.BlockSpec((1, 1), lambda i: (0, 0)),
        ],
        out_specs=pl.BlockSpec((G, 1), lambda i: (0, 0)),
        out_shape=jax.ShapeDtypeStruct((G, 1), f32),
        scratch_shapes=[pltpu.VMEM((G, H), f32)],
    )(p, p, w3, b3, bt, wf1, bf1, wf2, bf2)


def kernel(x, edge_index, edge_attr, batch, opt_edge, We, be, Weo, beo,
           Wl1, bl1, Wn1, bn1, Wl2, bl2, Wn2, bn2, Wl3, bl3, Wn3, bn3,
           Wf1, bf1, Wf2, bf2):
    src = edge_index[0]
    dst = edge_index[1]
    a = edge_attr[:, 0]
    o = opt_edge[:, 0]
    reci = jnp.stack([src, dst])
    recf = jnp.stack([a, o])
    xf = x[:, 0]

    we = We[0]
    weo = Weo[0]
    p1 = we @ Wl1[:64, 0]
    q1 = weo @ Wl1[64:, 0]
    r1 = be @ Wl1[:64, 0] + beo @ Wl1[64:, 0] + bl1[0]
    coef1 = jnp.stack([jnp.full((16,), p1, f32),
                       jnp.full((16,), q1, f32),
                       jnp.full((16,), r1, f32)])
    w1 = Wn1[0]
    u2 = we @ Wl2[:64]
    v2 = weo @ Wl2[64:]
    cc2 = be @ Wl2[:64] + beo @ Wl2[64:] + bl2 + bn1
    coef2 = jnp.stack([w1, u2, v2, cc2, bn1])
    u3 = we @ Wl3[:64]
    v3 = weo @ Wl3[64:]
    c3 = be @ Wl3[:64] + beo @ Wl3[64:] + bl3
    coef3 = jnp.stack([u3, v3, c3])

    s = _k1(reci, recf, xf, coef1)
    p2 = _k2(reci, recf, s, coef2)
    h2 = _mm(p2, Wn2, bn2.reshape(1, H))
    p3 = _k4(reci, recf, h2, coef3)
    return _k5(p3, Wn3, bn3.reshape(1, H), batch.reshape(N // BLK, 1, BLK),
               Wf1, bf1.reshape(1, H), Wf2, bf2.reshape(1, 1))
